# baseline (speedup 1.0000x reference)
import dataclasses
import functools

import jax
import jax.numpy as jnp
from jax import lax
from jax.experimental import pallas as pl
from jax.experimental.pallas import tpu as pltpu
from jax.experimental.pallas import tpu_sc as plsc

NC, NS, L = 2, 16, 16


def _sc_compiler_params():
    cp = pltpu.CompilerParams()
    if "needs_layout_passes" in pltpu.CompilerParams.__dataclass_fields__:
        cp = dataclasses.replace(cp, needs_layout_passes=False)
    return cp


def _make_deg_kernel(E, N):
    NW = NC * NS
    EPW = E // NW
    NV = EPW // L
    REM = EPW - NV * L
    mesh = plsc.VectorSubcoreMesh(core_axis_name="c", subcore_axis_name="s")

    @functools.partial(
        pl.kernel,
        out_type=jax.ShapeDtypeStruct((NW, N), jnp.float32),
        mesh=mesh,
        compiler_params=_sc_compiler_params(),
        scratch_types=[
            pltpu.VMEM((EPW + L,), jnp.int32),
            pltpu.VMEM((N,), jnp.float32),
        ],
    )
    def deg_kernel(dst_hbm, out_hbm, idx_v, hist_v):
        wid = lax.axis_index("s") * NC + lax.axis_index("c")
        base = wid * EPW
        pltpu.sync_copy(dst_hbm.at[pl.ds(base, EPW)], idx_v.at[pl.ds(0, EPW)])
        zf = jnp.zeros((L,), jnp.float32)
        idx_v[pl.ds(EPW, L)] = jnp.zeros((L,), jnp.int32)

        @pl.loop(0, N, step=L)
        def _(i):
            hist_v[pl.ds(i, L)] = zf

        ones = jnp.ones((L,), jnp.float32)

        @pl.loop(0, NV * L, step=L)
        def _(i):
            plsc.addupdate_scatter(hist_v, [idx_v[pl.ds(i, L)]], ones)

        if REM:
            mask = lax.iota(jnp.int32, L) < REM
            plsc.addupdate_scatter(hist_v, [idx_v[pl.ds(NV * L, L)]], ones,
                                   mask=mask)
        pltpu.sync_copy(hist_v, out_hbm.at[wid])

    return deg_kernel


def _make_agg_kernel(EP, N, F):
    C = 128
    EPS = EP // NS
    NCH = EPS // C
    assert NCH % 2 == 0 and EPS % 8 == 0
    NACC = N + 8
    RPW = ((N + NS - 1) // NS + 7) // 8 * 8
    RPW_LAST = N - RPW * (NS - 1)
    assert RPW_LAST > 0 and RPW_LAST % 8 == 0
    mesh = plsc.VectorSubcoreMesh(core_axis_name="c", subcore_axis_name="s")

    @functools.partial(
        pl.kernel,
        out_type=[
            jax.ShapeDtypeStruct((N, F), jnp.float32),
            jax.ShapeDtypeStruct((N, F), jnp.float32),
        ],
        mesh=mesh,
        compiler_params=_sc_compiler_params(),
        scratch_types=(
            [pltpu.VMEM((C,), jnp.int32)] * 4
            + [pltpu.VMEM((C, F), jnp.float32)] * 2
            + [pltpu.VMEM_SHARED((NACC, F), jnp.float32)]
            + [pltpu.SemaphoreType.DMA] * 2
        ),
    )
    def agg_kernel(xs_a, xs_b, src_hbm, dst_hbm, zero_hbm, agg_a, agg_b,
                   sia, sib, dia, dib, rows_a, rows_b, acc, sema, semb):
        cid = lax.axis_index("c")
        sid = lax.axis_index("s")
        roff = pl.multiple_of(sid * RPW, 8)
        loff = pl.multiple_of((NS - 1) * RPW, 8)

        @pl.when(sid < NS - 1)
        def _():
            pltpu.sync_copy(zero_hbm.at[pl.ds(roff, RPW)],
                            acc.at[pl.ds(roff, RPW)])

        @pl.when(sid == NS - 1)
        def _():
            pltpu.sync_copy(zero_hbm.at[pl.ds(loff, RPW_LAST)],
                            acc.at[pl.ds(loff, RPW_LAST)])

        plsc.subcore_barrier()
        base = sid * EPS

        def run(table, out):
            def load_idx(j, si, di):
                off = base + j * C
                pltpu.sync_copy(src_hbm.at[pl.ds(off, C)], si)
                pltpu.sync_copy(dst_hbm.at[pl.ds(off, C)], di)

            load_idx(0, sia, dia)

            @pl.loop(0, NCH // 2)
            def _(q):
                j = q * 2
                ha = pltpu.async_copy(table.at[sia], rows_a, sema)
                load_idx(j + 1, sib, dib)
                ha.wait()
                pltpu.sync_copy(rows_a, acc.at[dia], add=True)

                hb = pltpu.async_copy(table.at[sib], rows_b, semb)
                load_idx(j + 2, sia, dia)
                hb.wait()
                pltpu.sync_copy(rows_b, acc.at[dib], add=True)

            plsc.subcore_barrier()

            @pl.when(sid < NS - 1)
            def _():
                pltpu.sync_copy(acc.at[pl.ds(roff, RPW)],
                                out.at[pl.ds(roff, RPW)])

            @pl.when(sid == NS - 1)
            def _():
                pltpu.sync_copy(acc.at[pl.ds(loff, RPW_LAST)],
                                out.at[pl.ds(loff, RPW_LAST)])

        @pl.when(cid == 0)
        def _():
            run(xs_a, agg_a)

        @pl.when(cid == 1)
        def _():
            run(xs_b, agg_b)

    return agg_kernel


def _prologue_call(x, degp, N, F):
    NW = degp.shape[0]

    def body(x_ref, degp_ref, a_ref, b_ref, dv_ref):
        ones = jnp.ones((NW, 1), jnp.float32)
        deg = lax.dot_general(degp_ref[...], ones, (((0,), (0,)), ((), ())),
                              preferred_element_type=jnp.float32) + 1.0
        dv = lax.rsqrt(deg)
        dv_ref[...] = dv
        xs = x_ref[...] * dv
        a_ref[...] = xs[:, :F]
        b_ref[...] = xs[:, F:]

    return pl.pallas_call(
        body,
        out_shape=[jax.ShapeDtypeStruct((N, F), jnp.float32),
                   jax.ShapeDtypeStruct((N, F), jnp.float32),
                   jax.ShapeDtypeStruct((N, 1), jnp.float32)])(x, degp)


def _epilogue_call(x, agg_a, agg_b, dinv_col, batch3, W1, b1, W2, b2,
                   N, R, G, HID):
    nblk = N // R

    def body(x_ref, aa_ref, ab_ref, dv_ref, b_ref, W1_ref, b1_ref, W2_ref,
             b2_ref, node_ref, graph_ref, sums_ref, cnts_ref):
        i = pl.program_id(0)
        dv = dv_ref[...]
        agg = jnp.concatenate([aa_ref[...], ab_ref[...]], axis=1)
        Z = dv * (agg + dv * x_ref[...])
        H = jnp.tanh(
            jnp.dot(Z, W1_ref[...], preferred_element_type=jnp.float32)
            + b1_ref[...])
        node_ref[...] = H
        bat = b_ref[0]
        gid = lax.broadcasted_iota(jnp.int32, (G, R), 0)
        onehot = (bat == gid).astype(jnp.float32)
        psum = jnp.dot(onehot, H, preferred_element_type=jnp.float32)
        pcnt = jnp.sum(onehot, axis=1, keepdims=True)

        @pl.when(i == 0)
        def _():
            sums_ref[...] = psum
            cnts_ref[...] = jnp.broadcast_to(pcnt, (G, 128))

        @pl.when(i > 0)
        def _():
            sums_ref[...] += psum
            cnts_ref[...] += jnp.broadcast_to(pcnt, (G, 128))

        @pl.when(i == nblk - 1)
        def _():
            cnt = jnp.maximum(cnts_ref[:, :1], 1.0)
            mean = sums_ref[...] / cnt
            graph_ref[...] = jnp.tanh(
                jnp.dot(mean, W2_ref[...], preferred_element_type=jnp.float32)
                + b2_ref[...])

    F = agg_a.shape[1]
    IN = x.shape[1]
    return pl.pallas_call(
        body,
        grid=(nblk,),
        in_specs=[
            pl.BlockSpec((R, IN), lambda i: (i, 0)),
            pl.BlockSpec((R, F), lambda i: (i, 0)),
            pl.BlockSpec((R, F), lambda i: (i, 0)),
            pl.BlockSpec((R, 1), lambda i: (i, 0)),
            pl.BlockSpec((1, 1, R), lambda i: (i, 0, 0)),
            pl.BlockSpec((IN, HID), lambda i: (0, 0)),
            pl.BlockSpec((1, HID), lambda i: (0, 0)),
            pl.BlockSpec((HID, HID), lambda i: (0, 0)),
            pl.BlockSpec((1, HID), lambda i: (0, 0)),
        ],
        out_specs=[
            pl.BlockSpec((R, HID), lambda i: (i, 0)),
            pl.BlockSpec((G, HID), lambda i: (0, 0)),
        ],
        out_shape=[jax.ShapeDtypeStruct((N, HID), jnp.float32),
                   jax.ShapeDtypeStruct((G, HID), jnp.float32)],
        scratch_shapes=[pltpu.VMEM((G, HID), jnp.float32),
                        pltpu.VMEM((G, 128), jnp.float32)],
    )(x, agg_a, agg_b, dinv_col, batch3, W1, b1, W2, b2)


def _impl(x, edge_index, batch, W1, b1, W2, b2):
    N, IN = x.shape
    E = edge_index.shape[1]
    HID = W1.shape[1]
    G = 64
    F = IN // 2
    R = 1000

    ei = edge_index.astype(jnp.int32)
    src, dst = ei[0], ei[1]

    degp = _make_deg_kernel(E, N)(dst)
    xs_a, xs_b, dinv_col = _prologue_call(x, degp, N, F)

    EP = ((E + 128 * NS * 2 - 1) // (128 * NS * 2)) * 128 * NS * 2
    src2 = jnp.concatenate([src, jnp.zeros((EP + 128 - E,), jnp.int32)])
    dst2 = jnp.concatenate([dst, jnp.full((EP + 128 - E,), N, jnp.int32)])
    zeros = jnp.zeros((N, F), jnp.float32)
    agg_a, agg_b = _make_agg_kernel(EP, N, F)(xs_a, xs_b, src2, dst2, zeros)

    batch3 = batch.astype(jnp.int32).reshape(N // R, 1, R)
    node, graph = _epilogue_call(
        x, agg_a, agg_b, dinv_col, batch3,
        W1, b1.reshape(1, HID), W2, b2.reshape(1, HID), N, R, G, HID)
    return (graph, node)


kernel = jax.jit(_impl)


# speedup vs baseline: 1.3973x; 1.3973x over previous
import dataclasses
import functools

import jax
import jax.numpy as jnp
from jax import lax
from jax.experimental import pallas as pl
from jax.experimental.pallas import tpu as pltpu
from jax.experimental.pallas import tpu_sc as plsc

NC, NS, L = 2, 16, 16


def _sc_compiler_params():
    cp = pltpu.CompilerParams()
    if "needs_layout_passes" in pltpu.CompilerParams.__dataclass_fields__:
        cp = dataclasses.replace(cp, needs_layout_passes=False)
    return cp


def _make_deg_kernel(E, N):
    NW = NC * NS
    EPW = E // NW
    NV = EPW // L
    REM = EPW - NV * L
    mesh = plsc.VectorSubcoreMesh(core_axis_name="c", subcore_axis_name="s")

    @functools.partial(
        pl.kernel,
        out_type=jax.ShapeDtypeStruct((NW, N), jnp.float32),
        mesh=mesh,
        compiler_params=_sc_compiler_params(),
        scratch_types=[
            pltpu.VMEM((EPW + L,), jnp.int32),
            pltpu.VMEM((N,), jnp.float32),
        ],
    )
    def deg_kernel(dst_hbm, out_hbm, idx_v, hist_v):
        wid = lax.axis_index("s") * NC + lax.axis_index("c")
        base = wid * EPW
        pltpu.sync_copy(dst_hbm.at[pl.ds(base, EPW)], idx_v.at[pl.ds(0, EPW)])
        zf = jnp.zeros((L,), jnp.float32)
        idx_v[pl.ds(EPW, L)] = jnp.zeros((L,), jnp.int32)

        @pl.loop(0, N, step=L)
        def _(i):
            hist_v[pl.ds(i, L)] = zf

        ones = jnp.ones((L,), jnp.float32)

        @pl.loop(0, NV * L, step=L)
        def _(i):
            plsc.addupdate_scatter(hist_v, [idx_v[pl.ds(i, L)]], ones)

        if REM:
            mask = lax.iota(jnp.int32, L) < REM
            plsc.addupdate_scatter(hist_v, [idx_v[pl.ds(NV * L, L)]], ones,
                                   mask=mask)
        pltpu.sync_copy(hist_v, out_hbm.at[wid])

    return deg_kernel


def _make_agg_kernel(E, N, F):
    C = 128
    EPS = E // NS
    NCH = EPS // C
    REM = EPS - NCH * C
    assert EPS % 8 == 0
    RPW = ((N + NS - 1) // NS + 7) // 8 * 8
    RPW_LAST = N - RPW * (NS - 1)
    assert RPW_LAST > 0 and RPW_LAST % 8 == 0
    mesh = plsc.VectorSubcoreMesh(core_axis_name="c", subcore_axis_name="s")

    @functools.partial(
        pl.kernel,
        out_type=[
            jax.ShapeDtypeStruct((N, F), jnp.float32),
            jax.ShapeDtypeStruct((N, F), jnp.float32),
        ],
        mesh=mesh,
        compiler_params=_sc_compiler_params(),
        scratch_types=(
            [pltpu.VMEM((C,), jnp.int32)] * 2
            + [pltpu.VMEM((C, F), jnp.float32)]
            + ([pltpu.VMEM((REM,), jnp.int32)] * 2 if REM else [])
            + ([pltpu.VMEM((REM, F), jnp.float32)] if REM else [])
            + [pltpu.VMEM_SHARED((N, F), jnp.float32)]
            + [pltpu.SemaphoreType.DMA]
        ),
    )
    def agg_kernel(xs_a, xs_b, src_hbm, dst_hbm, zero_hbm, agg_a, agg_b,
                   *refs):
        if REM:
            sidx, didx, rows, sidx_r, didx_r, rows_r, acc, sem = refs
        else:
            sidx, didx, rows, acc, sem = refs
        cid = lax.axis_index("c")
        sid = lax.axis_index("s")
        roff = pl.multiple_of(sid * RPW, 8)
        loff = pl.multiple_of((NS - 1) * RPW, 8)

        @pl.when(sid < NS - 1)
        def _():
            pltpu.sync_copy(zero_hbm.at[pl.ds(roff, RPW)],
                            acc.at[pl.ds(roff, RPW)])

        @pl.when(sid == NS - 1)
        def _():
            pltpu.sync_copy(zero_hbm.at[pl.ds(loff, RPW_LAST)],
                            acc.at[pl.ds(loff, RPW_LAST)])

        plsc.subcore_barrier()
        base = sid * EPS

        def run(table, out):
            @pl.loop(0, NCH)
            def _(i):
                off = base + i * C
                pltpu.sync_copy(src_hbm.at[pl.ds(off, C)], sidx)
                pltpu.sync_copy(dst_hbm.at[pl.ds(off, C)], didx)
                pltpu.async_copy(table.at[sidx], rows, sem).wait()
                pltpu.sync_copy(rows, acc.at[didx], add=True)

            if REM:
                off = base + NCH * C
                pltpu.sync_copy(src_hbm.at[pl.ds(off, REM)], sidx_r)
                pltpu.sync_copy(dst_hbm.at[pl.ds(off, REM)], didx_r)
                pltpu.async_copy(table.at[sidx_r], rows_r, sem).wait()
                pltpu.sync_copy(rows_r, acc.at[didx_r], add=True)

            plsc.subcore_barrier()

            @pl.when(sid < NS - 1)
            def _():
                pltpu.sync_copy(acc.at[pl.ds(roff, RPW)],
                                out.at[pl.ds(roff, RPW)])

            @pl.when(sid == NS - 1)
            def _():
                pltpu.sync_copy(acc.at[pl.ds(loff, RPW_LAST)],
                                out.at[pl.ds(loff, RPW_LAST)])

        @pl.when(cid == 0)
        def _():
            run(xs_a, agg_a)

        @pl.when(cid == 1)
        def _():
            run(xs_b, agg_b)

    return agg_kernel


def _prologue_call(x, degp, N, F):
    NW = degp.shape[0]

    def body(x_ref, degp_ref, a_ref, b_ref, dv_ref):
        ones = jnp.ones((NW, 1), jnp.float32)
        deg = lax.dot_general(degp_ref[...], ones, (((0,), (0,)), ((), ())),
                              preferred_element_type=jnp.float32) + 1.0
        dv = lax.rsqrt(deg)
        dv_ref[...] = dv
        xs = x_ref[...] * dv
        a_ref[...] = xs[:, :F]
        b_ref[...] = xs[:, F:]

    return pl.pallas_call(
        body,
        out_shape=[jax.ShapeDtypeStruct((N, F), jnp.float32),
                   jax.ShapeDtypeStruct((N, F), jnp.float32),
                   jax.ShapeDtypeStruct((N, 1), jnp.float32)])(x, degp)


def _epilogue_call(x, agg_a, agg_b, dinv_col, batch3, W1, b1, W2, b2,
                   N, R, G, HID):
    nblk = N // R

    def body(x_ref, aa_ref, ab_ref, dv_ref, b_ref, W1_ref, b1_ref, W2_ref,
             b2_ref, node_ref, graph_ref, sums_ref, cnts_ref):
        i = pl.program_id(0)
        dv = dv_ref[...]
        agg = jnp.concatenate([aa_ref[...], ab_ref[...]], axis=1)
        Z = dv * (agg + dv * x_ref[...])
        H = jnp.tanh(
            jnp.dot(Z, W1_ref[...], preferred_element_type=jnp.float32)
            + b1_ref[...])
        node_ref[...] = H
        bat = b_ref[0]
        gid = lax.broadcasted_iota(jnp.int32, (G, R), 0)
        onehot = (bat == gid).astype(jnp.float32)
        psum = jnp.dot(onehot, H, preferred_element_type=jnp.float32)
        pcnt = jnp.sum(onehot, axis=1, keepdims=True)

        @pl.when(i == 0)
        def _():
            sums_ref[...] = psum
            cnts_ref[...] = jnp.broadcast_to(pcnt, (G, 128))

        @pl.when(i > 0)
        def _():
            sums_ref[...] += psum
            cnts_ref[...] += jnp.broadcast_to(pcnt, (G, 128))

        @pl.when(i == nblk - 1)
        def _():
            cnt = jnp.maximum(cnts_ref[:, :1], 1.0)
            mean = sums_ref[...] / cnt
            graph_ref[...] = jnp.tanh(
                jnp.dot(mean, W2_ref[...], preferred_element_type=jnp.float32)
                + b2_ref[...])

    F = agg_a.shape[1]
    IN = x.shape[1]
    return pl.pallas_call(
        body,
        grid=(nblk,),
        in_specs=[
            pl.BlockSpec((R, IN), lambda i: (i, 0)),
            pl.BlockSpec((R, F), lambda i: (i, 0)),
            pl.BlockSpec((R, F), lambda i: (i, 0)),
            pl.BlockSpec((R, 1), lambda i: (i, 0)),
            pl.BlockSpec((1, 1, R), lambda i: (i, 0, 0)),
            pl.BlockSpec((IN, HID), lambda i: (0, 0)),
            pl.BlockSpec((1, HID), lambda i: (0, 0)),
            pl.BlockSpec((HID, HID), lambda i: (0, 0)),
            pl.BlockSpec((1, HID), lambda i: (0, 0)),
        ],
        out_specs=[
            pl.BlockSpec((R, HID), lambda i: (i, 0)),
            pl.BlockSpec((G, HID), lambda i: (0, 0)),
        ],
        out_shape=[jax.ShapeDtypeStruct((N, HID), jnp.float32),
                   jax.ShapeDtypeStruct((G, HID), jnp.float32)],
        scratch_shapes=[pltpu.VMEM((G, HID), jnp.float32),
                        pltpu.VMEM((G, 128), jnp.float32)],
    )(x, agg_a, agg_b, dinv_col, batch3, W1, b1, W2, b2)


def _impl(x, edge_index, batch, W1, b1, W2, b2):
    N, IN = x.shape
    E = edge_index.shape[1]
    HID = W1.shape[1]
    G = 64
    F = IN // 2
    R = 1000

    ei = edge_index.astype(jnp.int32)
    src, dst = ei[0], ei[1]

    degp = _make_deg_kernel(E, N)(dst)
    xs_a, xs_b, dinv_col = _prologue_call(x, degp, N, F)

    zeros = jnp.zeros((N, F), jnp.float32)
    agg_a, agg_b = _make_agg_kernel(E, N, F)(xs_a, xs_b, src, dst, zeros)

    batch3 = batch.astype(jnp.int32).reshape(N // R, 1, R)
    node, graph = _epilogue_call(
        x, agg_a, agg_b, dinv_col, batch3,
        W1, b1.reshape(1, HID), W2, b2.reshape(1, HID), N, R, G, HID)
    return (graph, node)


kernel = jax.jit(_impl)


# speedup vs baseline: 1.5800x; 1.1308x over previous
import dataclasses
import functools

import jax
import jax.numpy as jnp
from jax import lax
from jax.experimental import pallas as pl
from jax.experimental.pallas import tpu as pltpu
from jax.experimental.pallas import tpu_sc as plsc

NC, NS, L = 2, 16, 16


def _sc_compiler_params():
    cp = pltpu.CompilerParams()
    if "needs_layout_passes" in pltpu.CompilerParams.__dataclass_fields__:
        cp = dataclasses.replace(cp, needs_layout_passes=False)
    return cp


def _make_deg_kernel(E, N):
    NW = NC * NS
    EPW = E // NW
    NV = EPW // L
    REM = EPW - NV * L
    mesh = plsc.VectorSubcoreMesh(core_axis_name="c", subcore_axis_name="s")

    @functools.partial(
        pl.kernel,
        out_type=jax.ShapeDtypeStruct((NW, N), jnp.float32),
        mesh=mesh,
        compiler_params=_sc_compiler_params(),
        scratch_types=[
            pltpu.VMEM((EPW + L,), jnp.int32),
            pltpu.VMEM((N,), jnp.float32),
        ],
    )
    def deg_kernel(dst_hbm, out_hbm, idx_v, hist_v):
        wid = lax.axis_index("s") * NC + lax.axis_index("c")
        base = wid * EPW
        pltpu.sync_copy(dst_hbm.at[pl.ds(base, EPW)], idx_v.at[pl.ds(0, EPW)])
        zf = jnp.zeros((L,), jnp.float32)
        idx_v[pl.ds(EPW, L)] = jnp.zeros((L,), jnp.int32)

        @pl.loop(0, N, step=L)
        def _(i):
            hist_v[pl.ds(i, L)] = zf

        ones = jnp.ones((L,), jnp.float32)

        @pl.loop(0, NV * L, step=L)
        def _(i):
            plsc.addupdate_scatter(hist_v, [idx_v[pl.ds(i, L)]], ones)

        if REM:
            mask = lax.iota(jnp.int32, L) < REM
            plsc.addupdate_scatter(hist_v, [idx_v[pl.ds(NV * L, L)]], ones,
                                   mask=mask)
        pltpu.sync_copy(hist_v, out_hbm.at[wid])

    return deg_kernel


def _make_agg_kernel(E, N, F):
    C = 128
    EPS = E // NS
    NCH = EPS // C
    REM = EPS - NCH * C
    assert EPS % 8 == 0
    RPW = ((N + NS - 1) // NS + 7) // 8 * 8
    RPW_LAST = N - RPW * (NS - 1)
    assert RPW_LAST > 0 and RPW_LAST % 8 == 0
    mesh = plsc.VectorSubcoreMesh(core_axis_name="c", subcore_axis_name="s")

    @functools.partial(
        pl.kernel,
        out_type=[
            jax.ShapeDtypeStruct((N, F), jnp.float32),
            jax.ShapeDtypeStruct((N, F), jnp.float32),
        ],
        mesh=mesh,
        compiler_params=_sc_compiler_params(),
        scratch_types=(
            [pltpu.VMEM((C,), jnp.int32)] * 2
            + [pltpu.VMEM((C, F), jnp.float32)]
            + ([pltpu.VMEM((REM,), jnp.int32)] * 2 if REM else [])
            + ([pltpu.VMEM((REM, F), jnp.float32)] if REM else [])
            + [pltpu.VMEM_SHARED((N, F), jnp.float32)]
            + [pltpu.SemaphoreType.DMA]
        ),
    )
    def agg_kernel(xs_a, xs_b, src_hbm, dst_hbm, zero_hbm, agg_a, agg_b,
                   *refs):
        if REM:
            sidx, didx, rows, sidx_r, didx_r, rows_r, acc, sem = refs
        else:
            sidx, didx, rows, acc, sem = refs
        cid = lax.axis_index("c")
        sid = lax.axis_index("s")
        roff = pl.multiple_of(sid * RPW, 8)
        loff = pl.multiple_of((NS - 1) * RPW, 8)

        @pl.when(sid < NS - 1)
        def _():
            pltpu.sync_copy(zero_hbm.at[pl.ds(roff, RPW)],
                            acc.at[pl.ds(roff, RPW)])

        @pl.when(sid == NS - 1)
        def _():
            pltpu.sync_copy(zero_hbm.at[pl.ds(loff, RPW_LAST)],
                            acc.at[pl.ds(loff, RPW_LAST)])

        plsc.subcore_barrier()
        base = sid * EPS

        def run(table, out):
            @pl.loop(0, NCH)
            def _(i):
                off = base + i * C
                pltpu.sync_copy(src_hbm.at[pl.ds(off, C)], sidx)
                h = pltpu.async_copy(table.at[sidx], rows, sem)
                pltpu.sync_copy(dst_hbm.at[pl.ds(off, C)], didx)
                h.wait()
                pltpu.sync_copy(rows, acc.at[didx], add=True)

            if REM:
                off = base + NCH * C
                pltpu.sync_copy(src_hbm.at[pl.ds(off, REM)], sidx_r)
                pltpu.sync_copy(dst_hbm.at[pl.ds(off, REM)], didx_r)
                pltpu.async_copy(table.at[sidx_r], rows_r, sem).wait()
                pltpu.sync_copy(rows_r, acc.at[didx_r], add=True)

            plsc.subcore_barrier()

            @pl.when(sid < NS - 1)
            def _():
                pltpu.sync_copy(acc.at[pl.ds(roff, RPW)],
                                out.at[pl.ds(roff, RPW)])

            @pl.when(sid == NS - 1)
            def _():
                pltpu.sync_copy(acc.at[pl.ds(loff, RPW_LAST)],
                                out.at[pl.ds(loff, RPW_LAST)])

        @pl.when(cid == 0)
        def _():
            run(xs_a, agg_a)

        @pl.when(cid == 1)
        def _():
            run(xs_b, agg_b)

    return agg_kernel


def _prologue_call(x, degp, N, F):
    NW = degp.shape[0]

    def body(x_ref, degp_ref, a_ref, b_ref, dv_ref):
        ones = jnp.ones((NW, 1), jnp.float32)
        deg = lax.dot_general(degp_ref[...], ones, (((0,), (0,)), ((), ())),
                              preferred_element_type=jnp.float32) + 1.0
        dv = lax.rsqrt(deg)
        dv_ref[...] = dv
        xs = x_ref[...] * dv
        a_ref[...] = xs[:, :F]
        b_ref[...] = xs[:, F:]

    return pl.pallas_call(
        body,
        out_shape=[jax.ShapeDtypeStruct((N, F), jnp.float32),
                   jax.ShapeDtypeStruct((N, F), jnp.float32),
                   jax.ShapeDtypeStruct((N, 1), jnp.float32)])(x, degp)


def _epilogue_call(x, agg_a, agg_b, dinv_col, batch3, W1, b1, W2, b2,
                   N, R, G, HID):
    nblk = N // R

    def body(x_ref, aa_ref, ab_ref, dv_ref, b_ref, W1_ref, b1_ref, W2_ref,
             b2_ref, node_ref, graph_ref, sums_ref, cnts_ref):
        i = pl.program_id(0)
        dv = dv_ref[...]
        agg = jnp.concatenate([aa_ref[...], ab_ref[...]], axis=1)
        Z = dv * (agg + dv * x_ref[...])
        H = jnp.tanh(
            jnp.dot(Z, W1_ref[...], preferred_element_type=jnp.float32)
            + b1_ref[...])
        node_ref[...] = H
        bat = b_ref[0]
        gid = lax.broadcasted_iota(jnp.int32, (G, R), 0)
        onehot = (bat == gid).astype(jnp.float32)
        psum = jnp.dot(onehot, H, preferred_element_type=jnp.float32)
        pcnt = jnp.sum(onehot, axis=1, keepdims=True)

        @pl.when(i == 0)
        def _():
            sums_ref[...] = psum
            cnts_ref[...] = jnp.broadcast_to(pcnt, (G, 128))

        @pl.when(i > 0)
        def _():
            sums_ref[...] += psum
            cnts_ref[...] += jnp.broadcast_to(pcnt, (G, 128))

        @pl.when(i == nblk - 1)
        def _():
            cnt = jnp.maximum(cnts_ref[:, :1], 1.0)
            mean = sums_ref[...] / cnt
            graph_ref[...] = jnp.tanh(
                jnp.dot(mean, W2_ref[...], preferred_element_type=jnp.float32)
                + b2_ref[...])

    F = agg_a.shape[1]
    IN = x.shape[1]
    return pl.pallas_call(
        body,
        grid=(nblk,),
        in_specs=[
            pl.BlockSpec((R, IN), lambda i: (i, 0)),
            pl.BlockSpec((R, F), lambda i: (i, 0)),
            pl.BlockSpec((R, F), lambda i: (i, 0)),
            pl.BlockSpec((R, 1), lambda i: (i, 0)),
            pl.BlockSpec((1, 1, R), lambda i: (i, 0, 0)),
            pl.BlockSpec((IN, HID), lambda i: (0, 0)),
            pl.BlockSpec((1, HID), lambda i: (0, 0)),
            pl.BlockSpec((HID, HID), lambda i: (0, 0)),
            pl.BlockSpec((1, HID), lambda i: (0, 0)),
        ],
        out_specs=[
            pl.BlockSpec((R, HID), lambda i: (i, 0)),
            pl.BlockSpec((G, HID), lambda i: (0, 0)),
        ],
        out_shape=[jax.ShapeDtypeStruct((N, HID), jnp.float32),
                   jax.ShapeDtypeStruct((G, HID), jnp.float32)],
        scratch_shapes=[pltpu.VMEM((G, HID), jnp.float32),
                        pltpu.VMEM((G, 128), jnp.float32)],
    )(x, agg_a, agg_b, dinv_col, batch3, W1, b1, W2, b2)


def _impl(x, edge_index, batch, W1, b1, W2, b2):
    N, IN = x.shape
    E = edge_index.shape[1]
    HID = W1.shape[1]
    G = 64
    F = IN // 2
    R = 1000

    ei = edge_index.astype(jnp.int32)
    src, dst = ei[0], ei[1]

    degp = _make_deg_kernel(E, N)(dst)
    xs_a, xs_b, dinv_col = _prologue_call(x, degp, N, F)

    zeros = jnp.zeros((N, F), jnp.float32)
    agg_a, agg_b = _make_agg_kernel(E, N, F)(xs_a, xs_b, src, dst, zeros)

    batch3 = batch.astype(jnp.int32).reshape(N // R, 1, R)
    node, graph = _epilogue_call(
        x, agg_a, agg_b, dinv_col, batch3,
        W1, b1.reshape(1, HID), W2, b2.reshape(1, HID), N, R, G, HID)
    return (graph, node)


kernel = jax.jit(_impl)


# speedup vs baseline: 1.8152x; 1.1489x over previous
import dataclasses
import functools

import jax
import jax.numpy as jnp
from jax import lax
from jax.experimental import pallas as pl
from jax.experimental.pallas import tpu as pltpu
from jax.experimental.pallas import tpu_sc as plsc

NC, NS, L = 2, 16, 16


def _sc_compiler_params():
    cp = pltpu.CompilerParams()
    if "needs_layout_passes" in pltpu.CompilerParams.__dataclass_fields__:
        cp = dataclasses.replace(cp, needs_layout_passes=False)
    return cp


def _make_deg_kernel(E, N):
    NW = NC * NS
    EPW = E // NW
    NV = EPW // L
    REM = EPW - NV * L
    mesh = plsc.VectorSubcoreMesh(core_axis_name="c", subcore_axis_name="s")

    @functools.partial(
        pl.kernel,
        out_type=jax.ShapeDtypeStruct((NW, N), jnp.float32),
        mesh=mesh,
        compiler_params=_sc_compiler_params(),
        scratch_types=[
            pltpu.VMEM((EPW + L,), jnp.int32),
            pltpu.VMEM((N,), jnp.float32),
        ],
    )
    def deg_kernel(dst_hbm, out_hbm, idx_v, hist_v):
        wid = lax.axis_index("s") * NC + lax.axis_index("c")
        base = wid * EPW
        pltpu.sync_copy(dst_hbm.at[pl.ds(base, EPW)], idx_v.at[pl.ds(0, EPW)])
        zf = jnp.zeros((L,), jnp.float32)
        idx_v[pl.ds(EPW, L)] = jnp.zeros((L,), jnp.int32)

        @pl.loop(0, N, step=L)
        def _(i):
            hist_v[pl.ds(i, L)] = zf

        ones = jnp.ones((L,), jnp.float32)

        @pl.loop(0, NV * L, step=L)
        def _(i):
            plsc.addupdate_scatter(hist_v, [idx_v[pl.ds(i, L)]], ones)

        if REM:
            mask = lax.iota(jnp.int32, L) < REM
            plsc.addupdate_scatter(hist_v, [idx_v[pl.ds(NV * L, L)]], ones,
                                   mask=mask)
        pltpu.sync_copy(hist_v, out_hbm.at[wid])

    return deg_kernel


def _make_agg_kernel(E, N, F):
    C = 128
    EPS = E // NS
    NCH = EPS // C
    REM = EPS - NCH * C
    assert EPS % 8 == 0
    RPW = ((N + NS - 1) // NS + 7) // 8 * 8
    RPW_LAST = N - RPW * (NS - 1)
    assert RPW_LAST > 0 and RPW_LAST % 8 == 0
    mesh = plsc.VectorSubcoreMesh(core_axis_name="c", subcore_axis_name="s")

    @functools.partial(
        pl.kernel,
        out_type=[
            jax.ShapeDtypeStruct((N, F), jnp.float32),
            jax.ShapeDtypeStruct((N, F), jnp.float32),
        ],
        mesh=mesh,
        compiler_params=_sc_compiler_params(),
        scratch_types=(
            [pltpu.VMEM((C,), jnp.int32)] * 3
            + [pltpu.VMEM((C, F), jnp.float32)]
            + ([pltpu.VMEM((REM,), jnp.int32)] * 2 if REM else [])
            + ([pltpu.VMEM((REM, F), jnp.float32)] if REM else [])
            + [pltpu.VMEM_SHARED((N, F), jnp.float32)]
            + [pltpu.SemaphoreType.DMA]
        ),
    )
    def agg_kernel(xs_a, xs_b, src_hbm, dst_hbm, zero_hbm, agg_a, agg_b,
                   *refs):
        if REM:
            sia, sib, didx, rows, sidx_r, didx_r, rows_r, acc, sem = refs
        else:
            sia, sib, didx, rows, acc, sem = refs
        assert NCH % 2 == 0
        cid = lax.axis_index("c")
        sid = lax.axis_index("s")
        roff = pl.multiple_of(sid * RPW, 8)
        loff = pl.multiple_of((NS - 1) * RPW, 8)

        @pl.when(sid < NS - 1)
        def _():
            pltpu.sync_copy(zero_hbm.at[pl.ds(roff, RPW)],
                            acc.at[pl.ds(roff, RPW)])

        @pl.when(sid == NS - 1)
        def _():
            pltpu.sync_copy(zero_hbm.at[pl.ds(loff, RPW_LAST)],
                            acc.at[pl.ds(loff, RPW_LAST)])

        plsc.subcore_barrier()
        base = sid * EPS

        def run(table, out):
            pltpu.sync_copy(src_hbm.at[pl.ds(base, C)], sia)

            @pl.loop(0, NCH // 2)
            def _(q):
                j = q * 2
                h = pltpu.async_copy(table.at[sia], rows, sem)
                pltpu.sync_copy(dst_hbm.at[pl.ds(base + j * C, C)], didx)
                pltpu.sync_copy(src_hbm.at[pl.ds(base + (j + 1) * C, C)], sib)
                h.wait()
                pltpu.sync_copy(rows, acc.at[didx], add=True)

                h = pltpu.async_copy(table.at[sib], rows, sem)
                pltpu.sync_copy(dst_hbm.at[pl.ds(base + (j + 1) * C, C)],
                                didx)
                pltpu.sync_copy(src_hbm.at[pl.ds(base + (j + 2) * C, C)], sia)
                h.wait()
                pltpu.sync_copy(rows, acc.at[didx], add=True)

            if REM:
                off = base + NCH * C
                pltpu.sync_copy(src_hbm.at[pl.ds(off, REM)], sidx_r)
                pltpu.sync_copy(dst_hbm.at[pl.ds(off, REM)], didx_r)
                pltpu.async_copy(table.at[sidx_r], rows_r, sem).wait()
                pltpu.sync_copy(rows_r, acc.at[didx_r], add=True)

            plsc.subcore_barrier()

            @pl.when(sid < NS - 1)
            def _():
                pltpu.sync_copy(acc.at[pl.ds(roff, RPW)],
                                out.at[pl.ds(roff, RPW)])

            @pl.when(sid == NS - 1)
            def _():
                pltpu.sync_copy(acc.at[pl.ds(loff, RPW_LAST)],
                                out.at[pl.ds(loff, RPW_LAST)])

        @pl.when(cid == 0)
        def _():
            run(xs_a, agg_a)

        @pl.when(cid == 1)
        def _():
            run(xs_b, agg_b)

    return agg_kernel


def _prologue_call(x, degp, N, F):
    NW = degp.shape[0]

    def body(x_ref, degp_ref, a_ref, b_ref, dv_ref):
        ones = jnp.ones((NW, 1), jnp.float32)
        deg = lax.dot_general(degp_ref[...], ones, (((0,), (0,)), ((), ())),
                              preferred_element_type=jnp.float32) + 1.0
        dv = lax.rsqrt(deg)
        dv_ref[...] = dv
        xs = x_ref[...] * dv
        a_ref[...] = xs[:, :F]
        b_ref[...] = xs[:, F:]

    return pl.pallas_call(
        body,
        out_shape=[jax.ShapeDtypeStruct((N, F), jnp.float32),
                   jax.ShapeDtypeStruct((N, F), jnp.float32),
                   jax.ShapeDtypeStruct((N, 1), jnp.float32)])(x, degp)


def _epilogue_call(x, agg_a, agg_b, dinv_col, batch3, W1, b1, W2, b2,
                   N, R, G, HID):
    nblk = N // R

    def body(x_ref, aa_ref, ab_ref, dv_ref, b_ref, W1_ref, b1_ref, W2_ref,
             b2_ref, node_ref, graph_ref, sums_ref, cnts_ref):
        i = pl.program_id(0)
        dv = dv_ref[...]
        agg = jnp.concatenate([aa_ref[...], ab_ref[...]], axis=1)
        Z = dv * (agg + dv * x_ref[...])
        H = jnp.tanh(
            jnp.dot(Z, W1_ref[...], preferred_element_type=jnp.float32)
            + b1_ref[...])
        node_ref[...] = H
        bat = b_ref[0]
        gid = lax.broadcasted_iota(jnp.int32, (G, R), 0)
        onehot = (bat == gid).astype(jnp.float32)
        psum = jnp.dot(onehot, H, preferred_element_type=jnp.float32)
        pcnt = jnp.sum(onehot, axis=1, keepdims=True)

        @pl.when(i == 0)
        def _():
            sums_ref[...] = psum
            cnts_ref[...] = jnp.broadcast_to(pcnt, (G, 128))

        @pl.when(i > 0)
        def _():
            sums_ref[...] += psum
            cnts_ref[...] += jnp.broadcast_to(pcnt, (G, 128))

        @pl.when(i == nblk - 1)
        def _():
            cnt = jnp.maximum(cnts_ref[:, :1], 1.0)
            mean = sums_ref[...] / cnt
            graph_ref[...] = jnp.tanh(
                jnp.dot(mean, W2_ref[...], preferred_element_type=jnp.float32)
                + b2_ref[...])

    F = agg_a.shape[1]
    IN = x.shape[1]
    return pl.pallas_call(
        body,
        grid=(nblk,),
        in_specs=[
            pl.BlockSpec((R, IN), lambda i: (i, 0)),
            pl.BlockSpec((R, F), lambda i: (i, 0)),
            pl.BlockSpec((R, F), lambda i: (i, 0)),
            pl.BlockSpec((R, 1), lambda i: (i, 0)),
            pl.BlockSpec((1, 1, R), lambda i: (i, 0, 0)),
            pl.BlockSpec((IN, HID), lambda i: (0, 0)),
            pl.BlockSpec((1, HID), lambda i: (0, 0)),
            pl.BlockSpec((HID, HID), lambda i: (0, 0)),
            pl.BlockSpec((1, HID), lambda i: (0, 0)),
        ],
        out_specs=[
            pl.BlockSpec((R, HID), lambda i: (i, 0)),
            pl.BlockSpec((G, HID), lambda i: (0, 0)),
        ],
        out_shape=[jax.ShapeDtypeStruct((N, HID), jnp.float32),
                   jax.ShapeDtypeStruct((G, HID), jnp.float32)],
        scratch_shapes=[pltpu.VMEM((G, HID), jnp.float32),
                        pltpu.VMEM((G, 128), jnp.float32)],
    )(x, agg_a, agg_b, dinv_col, batch3, W1, b1, W2, b2)


def _impl(x, edge_index, batch, W1, b1, W2, b2):
    N, IN = x.shape
    E = edge_index.shape[1]
    HID = W1.shape[1]
    G = 64
    F = IN // 2
    R = 1000

    ei = edge_index.astype(jnp.int32)
    src, dst = ei[0], ei[1]

    degp = _make_deg_kernel(E, N)(dst)
    xs_a, xs_b, dinv_col = _prologue_call(x, degp, N, F)

    zeros = jnp.zeros((N, F), jnp.float32)
    src_pad = jnp.concatenate([src, jnp.zeros((128,), jnp.int32)])
    agg_a, agg_b = _make_agg_kernel(E, N, F)(xs_a, xs_b, src_pad, dst, zeros)

    batch3 = batch.astype(jnp.int32).reshape(N // R, 1, R)
    node, graph = _epilogue_call(
        x, agg_a, agg_b, dinv_col, batch3,
        W1, b1.reshape(1, HID), W2, b2.reshape(1, HID), N, R, G, HID)
    return (graph, node)


kernel = jax.jit(_impl)
